# Initial kernel scaffold; baseline (speedup 1.0000x reference)
import jax
import jax.numpy as jnp
from jax.experimental import pallas as pl


def kernel(atom_x, atom_edge_index, atom_batch, aa_x, aa_edge_index, aa_edge_attr, aa_batch, m2p_edge_index, Wd, asd, add_, bd, Wp, asp, adp, bp, Wep, aep, Wis, Wid, asi, adi, bi, wn1, bn1, wn2, bn2, Wr1, Wo1, bb1, Wr2, Wo2, bb2):
    raise NotImplementedError("write your pallas kernel here")


# speedup vs baseline: 21.4975x; 21.4975x over previous
import functools
import jax
import jax.numpy as jnp
from jax import lax
from jax.experimental import pallas as pl
from jax.experimental.pallas import tpu as pltpu
from jax.experimental.pallas import tpu_sc as plsc

H = 2
HID = 32
G = 256
EPS = 1e-5
OUT = H * HID

NC = 2
NS = 16
NW = NC * NS
CE = 80


def _leaky(x):
    return jnp.where(x >= 0, x, 0.2 * x)


def _elu(x):
    return jnp.where(x > 0, x, jnp.exp(jnp.minimum(x, 0.0)) - 1.0)


def _mm_body(x_ref, w_ref, o_ref, *, act):
    x = x_ref[:]
    if act == "elu":
        x = _elu(x)
    o_ref[:] = jnp.dot(x, w_ref[:], preferred_element_type=jnp.float32)


def _mm(x, w, block=1000, act=None):
    N, K = x.shape
    M = w.shape[1]
    assert N % block == 0, (N, block)
    return pl.pallas_call(
        functools.partial(_mm_body, act=act),
        grid=(N // block,),
        in_specs=[
            pl.BlockSpec((block, K), lambda i: (i, 0)),
            pl.BlockSpec((K, M), lambda i: (0, 0)),
        ],
        out_specs=pl.BlockSpec((block, M), lambda i: (i, 0)),
        out_shape=jax.ShapeDtypeStruct((N, M), jnp.float32),
    )(x, w)


def _seg_sum_body(b_ref, x_ref, o_ref):
    i = pl.program_id(0)
    seg = lax.broadcasted_iota(jnp.int32, (G, 1), 0)
    onehot = (b_ref[0] == seg).astype(jnp.float32)
    acc = jnp.dot(onehot, x_ref[:], preferred_element_type=jnp.float32)

    @pl.when(i == 0)
    def _():
        o_ref[:] = acc

    @pl.when(i > 0)
    def _():
        o_ref[:] = o_ref[:] + acc


def _seg_sum(x, batch2d, block=1000):
    N, F = x.shape
    b3 = batch2d.reshape(N // block, 1, block)
    return pl.pallas_call(
        _seg_sum_body,
        grid=(N // block,),
        in_specs=[
            pl.BlockSpec((1, 1, block), lambda i: (i, 0, 0)),
            pl.BlockSpec((block, F), lambda i: (i, 0)),
        ],
        out_specs=pl.BlockSpec((G, F), lambda i: (0, 0)),
        out_shape=jax.ShapeDtypeStruct((G, F), jnp.float32),
    )(b3, x)


def _seg_max_body(b_ref, x_ref, o_ref):
    i = pl.program_id(0)
    seg = lax.broadcasted_iota(jnp.int32, (G, 1), 0)
    mask = b_ref[0] == seg
    vals = jnp.where(mask, x_ref[0], -3.0e38)
    acc = jnp.max(vals, axis=1, keepdims=True)

    @pl.when(i == 0)
    def _():
        o_ref[:] = acc

    @pl.when(i > 0)
    def _():
        o_ref[:] = jnp.maximum(o_ref[:], acc)


def _seg_max(score2d, batch2d, block=1000):
    N = score2d.shape[1]
    b3 = batch2d.reshape(N // block, 1, block)
    s3 = score2d.reshape(N // block, 1, block)
    return pl.pallas_call(
        _seg_max_body,
        grid=(N // block,),
        in_specs=[
            pl.BlockSpec((1, 1, block), lambda i: (i, 0, 0)),
            pl.BlockSpec((1, 1, block), lambda i: (i, 0, 0)),
        ],
        out_specs=pl.BlockSpec((G, 1), lambda i: (0, 0)),
        out_shape=jax.ShapeDtypeStruct((G, 1), jnp.float32),
    )(b3, s3)


def _seg_bcast_body(b_ref, v_ref, o_ref):
    seg = lax.broadcasted_iota(jnp.int32, (1, G), 1)
    onehot = (b_ref[:] == seg).astype(jnp.float32)
    o_ref[:] = jnp.dot(onehot, v_ref[:], preferred_element_type=jnp.float32)


def _seg_bcast(vals, batch_col, block=1000):
    N = batch_col.shape[0]
    F = vals.shape[1]
    return pl.pallas_call(
        _seg_bcast_body,
        grid=(N // block,),
        in_specs=[
            pl.BlockSpec((block, 1), lambda i: (i, 0)),
            pl.BlockSpec((G, F), lambda i: (0, 0)),
        ],
        out_specs=pl.BlockSpec((block, F), lambda i: (i, 0)),
        out_shape=jax.ShapeDtypeStruct((N, F), jnp.float32),
    )(batch_col, vals)


def _sc_mesh():
    return plsc.VectorSubcoreMesh(core_axis_name="c", subcore_axis_name="s")


def _stripes(Nd):
    rpt = (Nd // (NS * 8)) * 8
    return rpt, Nd - (NS - 1) * rpt


def _sh_zero(zer_h, sh, bb, s, rpt, rlast):
    pltpu.sync_copy(zer_h.at[pl.ds(0, rlast)], bb)

    @pl.when(s < NS - 1)
    def _():
        pltpu.sync_copy(bb.at[pl.ds(0, rpt)], sh.at[pl.ds(s * rpt, rpt)])

    @pl.when(s == NS - 1)
    def _():
        pltpu.sync_copy(bb, sh.at[pl.ds(s * rpt, rlast)])


def _sh_out(sh, out, doff_base, bb, s, rpt, rlast):
    @pl.when(s < NS - 1)
    def _():
        pltpu.sync_copy(sh.at[pl.ds(s * rpt, rpt)], bb.at[pl.ds(0, rpt)])
        pltpu.sync_copy(bb.at[pl.ds(0, rpt)],
                        out.at[pl.ds(doff_base + s * rpt, rpt)])

    @pl.when(s == NS - 1)
    def _():
        pltpu.sync_copy(sh.at[pl.ds(s * rpt, rlast)], bb)
        pltpu.sync_copy(bb, out.at[pl.ds(doff_base + s * rpt, rlast)])


def _sh_zero2(zer_h, sh, bb, s, rpt, rlast):
    pltpu.sync_copy(zer_h.at[pl.ds(0, 16)], bb)

    def blk(b, cc):
        pltpu.sync_copy(bb, sh.at[pl.ds(s * rpt + b * 16, 16)])
        return cc

    @pl.when(s < NS - 1)
    def _():
        lax.fori_loop(0, rpt // 16, blk, 0)

    @pl.when(s == NS - 1)
    def _():
        lax.fori_loop(0, rlast // 16, blk, 0)


def _sh_out2(sh, out, row_base, bb, s, rpt, rlast):
    def blk(b, cc):
        pltpu.sync_copy(sh.at[pl.ds(s * rpt + b * 16, 16)], bb)
        pltpu.sync_copy(bb, out.at[pl.ds(row_base + s * rpt + b * 16, 16)])
        return cc

    @pl.when(s < NS - 1)
    def _():
        lax.fori_loop(0, rpt // 16, blk, 0)

    @pl.when(s == NS - 1)
    def _():
        lax.fori_loop(0, rlast // 16, blk, 0)


@functools.lru_cache(maxsize=None)
def _make_edge_ex(E, Nd, has_ale):
    nchunks = E // CE
    iters = -(-nchunks // NW)
    CEH = CE * H
    rpt, rlast = _stripes(Nd * H)
    scratch = [
        pltpu.VMEM((CEH,), jnp.int32),
        pltpu.VMEM((CEH,), jnp.int32),
        pltpu.VMEM((CEH,), jnp.int32),
        pltpu.VMEM((CEH,), jnp.int32),
        pltpu.VMEM((CEH,), jnp.float32),
        pltpu.VMEM((CEH,), jnp.float32),
        pltpu.VMEM((CEH,), jnp.float32),
        pltpu.VMEM((CEH,), jnp.float32),
    ]
    if has_ale:
        scratch.append(pltpu.VMEM((CEH,), jnp.float32))
    scratch.append(pltpu.VMEM_SHARED((Nd * H,), jnp.float32))
    scratch.append(pltpu.VMEM((rlast,), jnp.float32))
    out_type = [
        jax.ShapeDtypeStruct((E * H,), jnp.float32),
        jax.ShapeDtypeStruct((NC * Nd * H,), jnp.float32),
    ]

    def body(*refs):
        if has_ale:
            (sals_h, dal_h, deni_h, als_h, dpk_h, ale_h, zer_h, ex_o, den_o,
             sals_v, dal_v, mi_v, deni_v, a_v, b_v, m_v, ex_v, ale_v,
             den_sh, bb_v) = refs
        else:
            (sals_h, dal_h, deni_h, als_h, dpk_h, zer_h, ex_o, den_o,
             sals_v, dal_v, mi_v, deni_v, a_v, b_v, m_v, ex_v, den_sh,
             bb_v) = refs
            ale_h = ale_v = None
        c = lax.axis_index("c")
        s = lax.axis_index("s")
        wid = s * NC + c
        _sh_zero(zer_h, den_sh, bb_v, s, rpt, rlast)
        plsc.subcore_barrier()

        def chunk(i, carry):
            k = wid + NW * i

            @pl.when(k < nchunks)
            def _():
                base = k * CEH
                pltpu.sync_copy(sals_h.at[pl.ds(base, CEH)], sals_v)
                pltpu.sync_copy(dal_h.at[pl.ds(base, CEH)], dal_v)
                pltpu.sync_copy(deni_h.at[pl.ds(base, CEH)], deni_v)
                if has_ale:
                    pltpu.sync_copy(ale_h.at[pl.ds(base, CEH)], ale_v)
                for j in range(CEH // 16):
                    mi_v[pl.ds(j * 16, 16)] = dal_v[pl.ds(j * 16, 16)] + H
                pltpu.sync_copy(als_h.at[sals_v], a_v)
                pltpu.sync_copy(dpk_h.at[dal_v], b_v)
                pltpu.sync_copy(dpk_h.at[mi_v], m_v)
                for j in range(CEH // 16):
                    a = a_v[pl.ds(j * 16, 16)] + b_v[pl.ds(j * 16, 16)]
                    if has_ale:
                        a = a + ale_v[pl.ds(j * 16, 16)]
                    ex16 = jnp.exp(_leaky(a) - m_v[pl.ds(j * 16, 16)])
                    ex_v[pl.ds(j * 16, 16)] = ex16
                pltpu.sync_copy(ex_v, ex_o.at[pl.ds(base, CEH)])
                pltpu.sync_copy(ex_v, den_sh.at[deni_v], add=True)
            return carry

        lax.fori_loop(0, iters, chunk, 0)
        plsc.subcore_barrier()
        _sh_out(den_sh, den_o, c * (Nd * H), bb_v, s, rpt, rlast)

    return pl.kernel(body, out_type=out_type, mesh=_sc_mesh(),
                     scratch_types=scratch)


@functools.lru_cache(maxsize=None)
def _make_edge_agg(E, Nd):
    nchunks = E // CE
    iters = -(-nchunks // NW)
    CEH = CE * H
    rpt, rlast = _stripes(Nd)
    scratch = [
        pltpu.VMEM((CE,), jnp.int32),
        pltpu.VMEM((CE,), jnp.int32),
        pltpu.VMEM((CEH,), jnp.int32),
        pltpu.VMEM((CEH,), jnp.float32),
        pltpu.VMEM((CEH,), jnp.float32),
        pltpu.VMEM((CEH,), jnp.float32),
        pltpu.VMEM((CE, 2 * OUT), jnp.float32),
        pltpu.VMEM_SHARED((Nd, 2 * OUT), jnp.float32),
        pltpu.VMEM((16, 2 * OUT), jnp.float32),
        pltpu.SemaphoreType.DMA,
    ]
    out_type = [jax.ShapeDtypeStruct((NC * Nd, 2 * OUT), jnp.float32)]

    def body(src_h, dst_h, deni_h, ex_h, den_h, hsrc_h, zer_h, out_o,
             src_v, dst_v, deni_v, ex_v, dpe_v, w_v, rows_v, out_sh, bb_v,
             sem):
        c = lax.axis_index("c")
        s = lax.axis_index("s")
        wid = s * NC + c
        _sh_zero2(zer_h, out_sh, bb_v, s, rpt, rlast)
        plsc.subcore_barrier()

        def chunk(i, carry):
            k = wid + NW * i

            @pl.when(k < nchunks)
            def _():
                base = k * CE
                pltpu.sync_copy(src_h.at[pl.ds(base, CE)], src_v)
                pltpu.sync_copy(dst_h.at[pl.ds(base, CE)], dst_v)
                pltpu.sync_copy(deni_h.at[pl.ds(base * H, CEH)], deni_v)
                pltpu.sync_copy(ex_h.at[pl.ds(base * H, CEH)], ex_v)
                pltpu.sync_copy(den_h.at[deni_v], dpe_v)
                pltpu.async_copy(hsrc_h.at[src_v], rows_v, sem).wait()
                for j in range(CEH // 16):
                    w_v[pl.ds(j * 16, 16)] = (
                        ex_v[pl.ds(j * 16, 16)] /
                        (dpe_v[pl.ds(j * 16, 16)] + 1e-16))

                def edge(e, c2):
                    for h in range(H):
                        wsp = w_v[pl.ds(e * H + h, 1)][0]
                        for cc in range(HID // 16):
                            col = (h * HID + cc * 16)
                            rows_v[e, pl.ds(col, 16)] = (
                                rows_v[e, pl.ds(col, 16)] * wsp)
                    return c2

                lax.fori_loop(0, CE, edge, 0)
                pltpu.sync_copy(rows_v, out_sh.at[dst_v], add=True)
            return carry

        lax.fori_loop(0, iters, chunk, 0)
        plsc.subcore_barrier()
        _sh_out2(out_sh, out_o, c * Nd, bb_v, s, rpt, rlast)

    return pl.kernel(body, out_type=out_type, mesh=_sc_mesh(),
                     scratch_types=scratch)


@functools.lru_cache(maxsize=None)
def _make_gsum(E, Ns, Nd, F):
    nchunks = E // CE
    iters = -(-nchunks // NW)
    rpt, rlast = _stripes(Nd)
    scratch = [
        pltpu.VMEM((CE,), jnp.int32),
        pltpu.VMEM((CE,), jnp.int32),
        pltpu.VMEM((CE, F), jnp.float32),
        pltpu.VMEM_SHARED((Nd, F), jnp.float32),
        pltpu.VMEM((16, F), jnp.float32),
        pltpu.SemaphoreType.DMA,
    ]
    out_type = [jax.ShapeDtypeStruct((NC * Nd, F), jnp.float32)]

    def body(src_h, dst_h, x_h, zer_h, out_o, src_v, dst_v, rows_v, out_sh,
             bb_v, sem):
        c = lax.axis_index("c")
        s = lax.axis_index("s")
        wid = s * NC + c
        _sh_zero2(zer_h, out_sh, bb_v, s, rpt, rlast)
        plsc.subcore_barrier()

        def chunk(i, carry):
            k = wid + NW * i

            @pl.when(k < nchunks)
            def _():
                base = k * CE
                pltpu.sync_copy(src_h.at[pl.ds(base, CE)], src_v)
                pltpu.sync_copy(dst_h.at[pl.ds(base, CE)], dst_v)
                pltpu.async_copy(x_h.at[src_v], rows_v, sem).wait()
                pltpu.sync_copy(rows_v, out_sh.at[dst_v], add=True)
            return carry

        lax.fori_loop(0, iters, chunk, 0)
        plsc.subcore_barrier()
        _sh_out2(out_sh, out_o, c * Nd, bb_v, s, rpt, rlast)

    return pl.kernel(body, out_type=out_type, mesh=_sc_mesh(),
                     scratch_types=scratch)


def _embed_head_vec(a):
    A = jnp.zeros((OUT, H), jnp.float32)
    for hh in range(H):
        A = A.at[hh * HID:(hh + 1) * HID, hh].set(a[hh])
    return A


def _dstpack(ald, shift):
    return jnp.concatenate([ald, _leaky(ald + shift[None, :])], axis=1)


def _combine(p, Nd):
    return p[:Nd] + p[Nd:]


def _graph_layer_norm(x, batch2d, batch_col, weight, bias):
    N, F = x.shape
    stats = _seg_sum(
        jnp.concatenate([x, x * x, jnp.ones((N, 1), jnp.float32)], axis=1),
        batch2d)
    sx = stats[:, :F].sum(-1)
    sxx = stats[:, F:2 * F].sum(-1)
    cnt = stats[:, 2 * F]
    norm = jnp.clip(cnt, 1.0, None) * F
    mean = sx / norm
    var = jnp.maximum(sxx / norm - mean * mean, 0.0)
    inv = 1.0 / jnp.sqrt(var + EPS)
    mb = _seg_bcast(jnp.stack([mean, inv], axis=1), batch_col)
    return (x - mb[:, 0:1]) * mb[:, 1:2] * weight + bias


def _sag_pool(x, src, dst, batch2d, batch_col, W_rel, W_root, b, zerF):
    N = x.shape[0]
    aggp = _make_gsum(src.shape[0], N, N, x.shape[1])(src, dst, x, zerF)[0]
    agg = _combine(aggp, N)
    score = _mm(jnp.concatenate([agg, x], axis=1),
                jnp.concatenate([W_rel, W_root], axis=0)) + b
    smax = _seg_max(score.reshape(1, N), batch2d)
    smax = jnp.where(smax < -1.0e37, 0.0, smax)
    ex = jnp.exp(score - _seg_bcast(smax, batch_col))
    den = _seg_sum(ex, batch2d)
    s = ex / (_seg_bcast(den, batch_col) + 1e-16)
    return x * s


def kernel(atom_x, atom_edge_index, atom_batch, aa_x, aa_edge_index,
           aa_edge_attr, aa_batch, m2p_edge_index, Wd, asd, add_, bd, Wp, asp,
           adp, bp, Wep, aep, Wis, Wid, asi, adi, bi, wn1, bn1, wn2, bn2,
           Wr1, Wo1, bb1, Wr2, Wo2, bb2):
    Na = atom_x.shape[0]
    Np = aa_x.shape[0]

    a_src = atom_edge_index[0].astype(jnp.int32)
    a_dst = atom_edge_index[1].astype(jnp.int32)
    p_src = aa_edge_index[0].astype(jnp.int32)
    p_dst = aa_edge_index[1].astype(jnp.int32)
    m_src = m2p_edge_index[0].astype(jnp.int32)
    m_dst = m2p_edge_index[1].astype(jnp.int32)

    ab2 = atom_batch.astype(jnp.int32).reshape(1, Na)
    abc = atom_batch.astype(jnp.int32).reshape(Na, 1)
    pb2 = aa_batch.astype(jnp.int32).reshape(1, Np)
    pbc = aa_batch.astype(jnp.int32).reshape(Np, 1)

    zden = jnp.zeros((Na * H,), jnp.float32)
    zer64 = jnp.zeros((Na, OUT), jnp.float32)
    zer128 = jnp.zeros((Na, 2 * OUT), jnp.float32)

    Asd = _embed_head_vec(asd)
    Add = _embed_head_vec(add_)
    Asp = _embed_head_vec(asp)
    Adp = _embed_head_vec(adp)
    Aep = _embed_head_vec(aep)
    Asi = _embed_head_vec(asi)
    Adi = _embed_head_vec(adi)

    Wcat_a = jnp.concatenate([Wd, Wd @ Asd, Wd @ Add, Wid @ Adi], axis=1)
    Wcat_p = jnp.concatenate(
        [Wp, Wp @ Asp, Wp @ Adp, Wis, Wis @ Asi, Wid @ Adi], axis=1)

    proj_a = _mm(atom_x, Wcat_a, act="elu")
    proj_p = _mm(aa_x, Wcat_p, act="elu")
    hd, als1, ald1, ald2 = (proj_a[:, :64], proj_a[:, 64:66],
                            proj_a[:, 66:68], proj_a[:, 68:70])
    hp = proj_p[:, :64]
    als3 = proj_p[:, 64:66]
    ald3 = proj_p[:, 66:68]
    his = proj_p[:, 68:132]
    als2 = proj_p[:, 132:134]
    ald4 = proj_p[:, 134:136]

    he_al = _mm(aa_edge_attr, Wep @ Aep)

    dpk1 = _dstpack(ald1, als1.max(0))
    dpk2 = _dstpack(ald2, als2.max(0))
    dpk3 = _dstpack(ald3, als3.max(0) + he_al.max(0))

    Ea, Ep, Em = a_src.shape[0], p_src.shape[0], m_src.shape[0]

    def _slot_idx(src, dst):
        E = src.shape[0]
        hh = jnp.tile(jnp.arange(H, dtype=jnp.int32), E)
        srcd = jnp.repeat(src, H)
        dstd = jnp.repeat(dst, H)
        return srcd * H + hh, dstd * (2 * H) + hh, dstd * H + hh

    sals1, dal1, deni1 = _slot_idx(a_src, a_dst)
    sals2, dal2, deni2 = _slot_idx(m_dst, m_src)
    sals3, dal3, deni3 = _slot_idx(p_src, p_dst)
    sals4, dal4, deni4 = _slot_idx(m_src, m_dst)

    ex1, den1p = _make_edge_ex(Ea, Na, False)(
        sals1, dal1, deni1, als1.reshape(-1), dpk1.reshape(-1), zden)
    ex2, den2p = _make_edge_ex(Em, Na, False)(
        sals2, dal2, deni2, als2.reshape(-1), dpk2.reshape(-1), zden)
    ex3, den3p = _make_edge_ex(Ep, Np, True)(
        sals3, dal3, deni3, als3.reshape(-1), dpk3.reshape(-1),
        he_al.reshape(-1), zden)
    den1 = den1p.reshape(NC, -1).sum(0)
    den2 = den2p.reshape(NC, -1).sum(0)
    den3 = den3p.reshape(NC, -1).sum(0)

    pad64 = lambda x: jnp.concatenate([x, jnp.zeros_like(x)], axis=1)
    o1p, = _make_edge_agg(Ea, Na)(a_src, a_dst, deni1, ex1, den1, pad64(hd),
                                  zer128)
    o2p, = _make_edge_agg(Em, Na)(m_dst, m_src, deni2, ex2, den2, pad64(his),
                                  zer128)
    o3p, = _make_edge_agg(Ep, Np)(p_src, p_dst, deni3, ex3, den3, pad64(hp),
                                  zer128)

    atom_cat = jnp.concatenate(
        [_combine(o1p, Na)[:, :OUT] + bd, _combine(o2p, Na)[:, :OUT] + bi],
        axis=1)
    atom_x2 = _elu(_graph_layer_norm(atom_cat, ab2, abc, wn1, bn1))

    proj_a2 = _mm(atom_x2, jnp.concatenate([Wis, Wis @ Asi], axis=1))
    his4 = proj_a2[:, :64]
    als4 = proj_a2[:, 64:66]
    dpk4 = _dstpack(ald4, als4.max(0))
    ex4, den4p = _make_edge_ex(Em, Np, False)(
        sals4, dal4, deni4, als4.reshape(-1), dpk4.reshape(-1), zden)
    den4 = den4p.reshape(NC, -1).sum(0)
    o4p, = _make_edge_agg(Em, Np)(m_src, m_dst, deni4, ex4, den4, pad64(his4),
                                  zer128)

    aa_cat = jnp.concatenate(
        [_combine(o3p, Np)[:, :OUT] + bp, _combine(o4p, Np)[:, :OUT] + bi],
        axis=1)
    aa_x2 = _elu(_graph_layer_norm(aa_cat, pb2, pbc, wn2, bn2))

    atom_p = _sag_pool(atom_x2, a_src, a_dst, ab2, abc, Wr1, Wo1, bb1, zer128)
    aa_p = _sag_pool(aa_x2, p_src, p_dst, pb2, pbc, Wr2, Wo2, bb2, zer128)
    drug_g = _seg_sum(atom_p, ab2)
    prot_g = _seg_sum(aa_p, pb2)
    return (atom_p, aa_p, drug_g, prot_g)
